# baseline (speedup 1.0000x reference)
import jax
import jax.numpy as jnp
from jax import lax
from jax.experimental import pallas as pl
from jax.experimental.pallas import tpu as pltpu
from jax.experimental.pallas import tpu_sc as plsc

_VOCAB = 1000
_B = 1024
_L = 50
_VP = 1024
_NC, _NS = 2, 16
_NW = _NC * _NS
_NBT = _B // 128
_NVQ = _VP // 128
_UNITS = _L * _NBT * _NVQ
_UPW = _UNITS // _NW
_OBS = 129
_VTAIL = _VOCAB - 7 * 128


def kernel(x, table):
    xt = jnp.transpose(x).reshape(_L * _B).astype(jnp.int32)
    tblP = jnp.pad(table, ((0, 0), (0, _VP - _VOCAB)))

    mesh = plsc.VectorSubcoreMesh(core_axis_name="c", subcore_axis_name="s")

    @pl.kernel(
        out_type=jax.ShapeDtypeStruct((_L, _VOCAB, _B), table.dtype),
        mesh=mesh,
        compiler_params=pltpu.CompilerParams(
            use_tc_tiling_on_sc=True, needs_layout_passes=False),
        scratch_types=[
            pltpu.VMEM((128, 128), jnp.float32),
            pltpu.VMEM((128, 128), jnp.float32),
            pltpu.VMEM((128, _OBS), jnp.float32),
            pltpu.VMEM((128, _OBS), jnp.float32),
            pltpu.VMEM((128,), jnp.int32),
            pltpu.VMEM((128,), jnp.int32),
            pltpu.SemaphoreType.DMA,
            pltpu.SemaphoreType.DMA,
            pltpu.SemaphoreType.DMA,
            pltpu.SemaphoreType.DMA,
            pltpu.SemaphoreType.DMA,
            pltpu.SemaphoreType.DMA,
        ],
    )
    def lookup_kernel(tbl_hbm, xt_hbm, q_hbm, g0, g1, ob0, ob1, i0, i1,
                      is0, is1, gs0, gs1, ws0, ws1):
        gbufs = (g0, g1)
        obufs = (ob0, ob1)
        ibufs = (i0, i1)
        isems = (is0, is1)
        gsems = (gs0, gs1)
        wsems = (ws0, ws1)
        wid = lax.axis_index("s") * _NC + lax.axis_index("c")
        u0 = wid * _UPW

        def params(u):
            l = lax.div(u, _NBT * _NVQ)
            r = u - l * (_NBT * _NVQ)
            bt = lax.div(r, _NVQ)
            vq = r - bt * _NVQ
            return l, bt, vq

        def idx_desc(u, p):
            l, bt, _ = params(u)
            return pltpu.make_async_copy(
                xt_hbm.at[pl.ds(l * _B + bt * 128, 128)], ibufs[p], isems[p])

        def gather_desc(u, p):
            _, _, vq = params(u)
            return pltpu.make_async_copy(
                tbl_hbm.at[:, pl.ds(vq * 128, 128)].at[ibufs[p]],
                gbufs[p], gsems[p])

        def write_desc(u, p):
            l, bt, vq = params(u)
            return pltpu.make_async_copy(
                obufs[p].at[pl.ds(0, 128), pl.ds(0, 128)],
                q_hbm.at[l].at[pl.ds(vq * 128, 128)].at[:, pl.ds(bt * 128, 128)],
                wsems[p])

        def write_desc_tail(u, p):
            l, bt, _ = params(u)
            return pltpu.make_async_copy(
                obufs[p].at[pl.ds(0, _VTAIL), pl.ds(0, 128)],
                q_hbm.at[l].at[pl.ds(7 * 128, _VTAIL)].at[:, pl.ds(bt * 128, 128)],
                wsems[p])

        def write_start(u, p):
            _, _, vq = params(u)

            @pl.when(vq == _NVQ - 1)
            def _():
                write_desc_tail(u, p).start()

            @pl.when(vq != _NVQ - 1)
            def _():
                write_desc(u, p).start()

        def write_wait(u, p):
            _, _, vq = params(u)

            @pl.when(vq == _NVQ - 1)
            def _():
                write_desc_tail(u, p).wait()

            @pl.when(vq != _NVQ - 1)
            def _():
                write_desc(u, p).wait()

        iota16 = lax.iota(jnp.int32, 16)
        rowvs = [iota16 + t * 16 for t in range(8)]

        def transpose(p):
            @pl.loop(0, 128)
            def _(b):
                bcol = jnp.full((16,), b, jnp.int32)
                src = gbufs[p].at[b]
                for t in range(8):
                    vals = src[pl.ds(t * 16, 16)]
                    plsc.store_scatter(obufs[p], [rowvs[t], bcol], vals)

        idx_desc(u0, 0).start()
        idx_desc(u0, 0).wait()
        gather_desc(u0, 0).start()
        idx_desc(u0 + 1, 1).start()

        @pl.loop(0, _UPW, step=2)
        def _(du):
            for p in range(2):
                uu = du + p
                u = u0 + uu
                pp = 1 - p
                gather_desc(u, p).wait()

                @pl.when(uu >= 2)
                def _():
                    write_wait(u - 2, p)

                @pl.when(uu + 1 < _UPW)
                def _():
                    idx_desc(u + 1, pp).wait()
                    gather_desc(u + 1, pp).start()

                @pl.when(uu + 2 < _UPW)
                def _():
                    idx_desc(u + 2, p).start()

                transpose(p)
                write_start(u, p)

        write_wait(u0 + _UPW - 2, 0)
        write_wait(u0 + _UPW - 1, 1)

    q = lookup_kernel(tblP, xt)
    return jnp.transpose(q, (2, 0, 1))


# speedup vs baseline: 2.0235x; 2.0235x over previous
import jax
import jax.numpy as jnp
from jax import lax
from jax.experimental import pallas as pl
from jax.experimental.pallas import tpu as pltpu
from jax.experimental.pallas import tpu_sc as plsc

_VOCAB = 1000
_B = 1024
_L = 50
_VP = 1024
_NC, _NS = 2, 16
_NW = _NC * _NS
_NSEG = _B // 16
_SPLIT = 29


def kernel(x, table):
    xt = jnp.transpose(x).reshape(_L * _B).astype(jnp.int32)
    tblT = jnp.pad(
        jnp.transpose(table), ((0, _VP - _VOCAB), (0, _VP - _VOCAB))
    ).reshape(_VP * _VP)

    mesh = plsc.VectorSubcoreMesh(core_axis_name="c", subcore_axis_name="s")

    @pl.kernel(
        out_type=jax.ShapeDtypeStruct((_L, _VOCAB, _B), table.dtype),
        mesh=mesh,
        compiler_params=pltpu.CompilerParams(
            use_tc_tiling_on_sc=True, needs_layout_passes=False),
        scratch_types=[
            pltpu.VMEM((32 * _VP,), jnp.float32),
            pltpu.VMEM((_L * _B,), jnp.int32),
            pltpu.VMEM((8, _B), jnp.float32),
            pltpu.VMEM((8, _B), jnp.float32),
            pltpu.SemaphoreType.DMA,
            pltpu.SemaphoreType.DMA,
        ],
    )
    def lookup_kernel(tbl_hbm, xt_hbm, q_hbm, tbl_v, idx_v, ob0, ob1, ws0, ws1):
        obufs = (ob0, ob1)
        wsems = (ws0, ws1)
        wid = lax.axis_index("s") * _NC + lax.axis_index("c")
        small = wid >= _SPLIT
        nvt = jnp.where(small, 3, 4)
        vt0 = jnp.where(small, 4 * _SPLIT + 3 * (wid - _SPLIT), 4 * wid)
        v0 = vt0 * 8

        pltpu.sync_copy(tbl_hbm.at[pl.ds(v0 * _VP, 32 * _VP)], tbl_v)
        pltpu.sync_copy(xt_hbm, idx_v)

        total = _L * nvt

        def compute(l, j, ob):
            rows = [tbl_v.at[pl.ds((j * 8 + vi) * _VP, _VP)] for vi in range(8)]

            @pl.loop(0, _NSEG, step=2)
            def _(s):
                for t in range(2):
                    idx16 = idx_v[pl.ds(l * _B + (s + t) * 16, 16)]
                    for vi in range(8):
                        vals = plsc.load_gather(rows[vi], [idx16])
                        ob[vi, pl.ds((s + t) * 16, 16)] = vals

        def write_desc(l, j, p):
            return pltpu.make_async_copy(
                obufs[p], q_hbm.at[l].at[pl.ds(v0 + j * 8, 8)], wsems[p])

        @pl.loop(0, total, step=2)
        def _(u):
            for p in range(2):
                uu = u + p
                l = lax.div(uu, nvt)
                j = uu - l * nvt

                @pl.when(uu >= 2)
                def _():
                    write_desc(l, j, p).wait()

                compute(l, j, obufs[p])
                write_desc(l, j, p).start()

        for p in range(2):
            lastu = total - 2 + p
            ll = lax.div(lastu, nvt)
            write_desc(ll, lastu - ll * nvt, p).wait()

    q = lookup_kernel(tblT, xt)
    return jnp.transpose(q, (2, 0, 1))
